# baseline (speedup 1.0000x reference)
import functools

import jax
import jax.numpy as jnp
from jax import lax
from jax.experimental import pallas as pl
from jax.experimental.pallas import tpu as pltpu
from jax.experimental.pallas import tpu_sc as plsc

B = 4
C_IN = 512
C_OUT = 384
D = 8192
NROWS_OUT = B * C_OUT
NW = 32
ROWS_PER_TILE = NROWS_OUT // NW
CHUNK = 4
NCHUNK = ROWS_PER_TILE // CHUNK
NBUF = 3
LANES = 16

_mesh = plsc.VectorSubcoreMesh(core_axis_name="c", subcore_axis_name="s")


@functools.partial(
    pl.kernel,
    mesh=_mesh,
    compiler_params=pltpu.CompilerParams(
        skip_device_barrier=True,
        disable_bounds_checks=True,
        disable_semaphore_checks=True,
    ),
    out_type=jax.ShapeDtypeStruct((NROWS_OUT, D), jnp.float32),
    scratch_types=[
        pltpu.VMEM((NCHUNK, CHUNK), jnp.int32),
        *[pltpu.VMEM((CHUNK, D), jnp.float32) for _ in range(NBUF)],
        pltpu.SemaphoreType.DMA,
        *[pltpu.SemaphoreType.DMA for _ in range(2 * NBUF)],
    ],
)
def _sc_gather(table_hbm, idx_hbm, out_hbm, idx2, *bufs_and_sems):
    bufs = bufs_and_sems[:NBUF]
    isem = bufs_and_sems[NBUF]
    gsems = bufs_and_sems[NBUF + 1:2 * NBUF + 1]
    ssems = bufs_and_sems[2 * NBUF + 1:]
    wid = lax.axis_index("s") * 2 + lax.axis_index("c")
    base = wid * ROWS_PER_TILE
    jstart = base % C_OUT
    row_off = (base // C_OUT) * C_IN
    batch_tbl = table_hbm.at[pl.ds(row_off, C_IN)]
    pltpu.async_copy(idx_hbm.at[jstart // ROWS_PER_TILE], idx2, isem).wait()

    gathers = [None] * NBUF
    scatters = [None] * NBUF

    gathers[0] = pltpu.async_copy(batch_tbl.at[idx2.at[0]], bufs[0], gsems[0])
    for c in range(NCHUNK):
        nxt = c + 1
        if nxt < NCHUNK:
            sn = nxt % NBUF
            if scatters[sn] is not None:
                scatters[sn].wait()
                scatters[sn] = None
            gathers[sn] = pltpu.async_copy(
                batch_tbl.at[idx2.at[nxt]], bufs[sn], gsems[sn])
        s = c % NBUF
        gathers[s].wait()
        scatters[s] = pltpu.async_copy(
            bufs[s], out_hbm.at[pl.ds(base + c * CHUNK, CHUNK)], ssems[s])
    for s in range(NBUF):
        if scatters[s] is not None:
            scatters[s].wait()


def kernel(x, indices):
    table = x.reshape(B * C_IN, D)
    out = _sc_gather(
        table, indices.reshape(C_OUT // ROWS_PER_TILE, NCHUNK, CHUNK))
    return out.reshape(B, C_OUT, D)


# speedup vs baseline: 1.0100x; 1.0100x over previous
import functools

import jax
import jax.numpy as jnp
from jax import lax
from jax.experimental import pallas as pl
from jax.experimental.pallas import tpu as pltpu
from jax.experimental.pallas import tpu_sc as plsc

B = 4
C_IN = 512
C_OUT = 384
D = 8192
NROWS_OUT = B * C_OUT
NW = 32
ROWS_PER_TILE = NROWS_OUT // NW
CHUNK = 2
NCHUNK = ROWS_PER_TILE // CHUNK
NBUF = 7
LANES = 16

_mesh = plsc.VectorSubcoreMesh(core_axis_name="c", subcore_axis_name="s")


@functools.partial(
    pl.kernel,
    mesh=_mesh,
    compiler_params=pltpu.CompilerParams(
        skip_device_barrier=True,
        disable_bounds_checks=True,
        disable_semaphore_checks=True,
    ),
    out_type=jax.ShapeDtypeStruct((NROWS_OUT, D), jnp.float32),
    scratch_types=[
        pltpu.VMEM((NCHUNK, CHUNK), jnp.int32),
        *[pltpu.VMEM((CHUNK, D), jnp.float32) for _ in range(NBUF)],
        pltpu.SemaphoreType.DMA,
        *[pltpu.SemaphoreType.DMA for _ in range(2 * NBUF)],
    ],
)
def _sc_gather(table_hbm, idx_hbm, out_hbm, idx2, *bufs_and_sems):
    bufs = bufs_and_sems[:NBUF]
    isem = bufs_and_sems[NBUF]
    gsems = bufs_and_sems[NBUF + 1:2 * NBUF + 1]
    ssems = bufs_and_sems[2 * NBUF + 1:]
    wid = lax.axis_index("s") * 2 + lax.axis_index("c")
    base = wid * ROWS_PER_TILE
    jstart = base % C_OUT
    row_off = (base // C_OUT) * C_IN
    batch_tbl = table_hbm.at[pl.ds(row_off, C_IN)]
    pltpu.async_copy(idx_hbm.at[jstart // ROWS_PER_TILE], idx2, isem).wait()

    gathers = [None] * NBUF
    scatters = [None] * NBUF

    gathers[0] = pltpu.async_copy(batch_tbl.at[idx2.at[0]], bufs[0], gsems[0])
    gathers[1] = pltpu.async_copy(batch_tbl.at[idx2.at[1]], bufs[1], gsems[1])
    for c in range(NCHUNK):
        nxt = c + 2
        if nxt < NCHUNK:
            sn = nxt % NBUF
            if scatters[sn] is not None:
                scatters[sn].wait()
                scatters[sn] = None
            gathers[sn] = pltpu.async_copy(
                batch_tbl.at[idx2.at[nxt]], bufs[sn], gsems[sn])
        s = c % NBUF
        gathers[s].wait()
        scatters[s] = pltpu.async_copy(
            bufs[s], out_hbm.at[pl.ds(base + c * CHUNK, CHUNK)], ssems[s])
    for s in range(NBUF):
        if scatters[s] is not None:
            scatters[s].wait()


def kernel(x, indices):
    table = x.reshape(B * C_IN, D)
    out = _sc_gather(
        table, indices.reshape(C_OUT // ROWS_PER_TILE, NCHUNK, CHUNK))
    return out.reshape(B, C_OUT, D)


# speedup vs baseline: 1.0178x; 1.0077x over previous
import functools

import jax
import jax.numpy as jnp
from jax import lax
from jax.experimental import pallas as pl
from jax.experimental.pallas import tpu as pltpu
from jax.experimental.pallas import tpu_sc as plsc

B = 4
C_IN = 512
C_OUT = 384
D = 8192
NROWS_OUT = B * C_OUT
NW = 32
ROWS_PER_TILE = NROWS_OUT // NW
CHUNK = 2
NCHUNK = ROWS_PER_TILE // CHUNK
NBUF = 7
LANES = 16

_mesh = plsc.VectorSubcoreMesh(core_axis_name="c", subcore_axis_name="s")


@functools.partial(
    pl.kernel,
    mesh=_mesh,
    compiler_params=pltpu.CompilerParams(
        skip_device_barrier=True,
        disable_bounds_checks=True,
        disable_semaphore_checks=True,
    ),
    out_type=jax.ShapeDtypeStruct((NROWS_OUT, D), jnp.float32),
    scratch_types=[
        pltpu.VMEM((NCHUNK, CHUNK), jnp.int32),
        *[pltpu.VMEM((CHUNK, D), jnp.float32) for _ in range(NBUF)],
        pltpu.SemaphoreType.DMA,
        *[pltpu.SemaphoreType.DMA for _ in range(2 * NBUF)],
    ],
)
def _sc_gather(table_hbm, idx_hbm, out_hbm, idx2, *bufs_and_sems):
    bufs = bufs_and_sems[:NBUF]
    isem = bufs_and_sems[NBUF]
    gsems = bufs_and_sems[NBUF + 1:2 * NBUF + 1]
    ssems = bufs_and_sems[2 * NBUF + 1:]
    wid = lax.axis_index("s") * 2 + lax.axis_index("c")
    base = wid * ROWS_PER_TILE
    jstart = base % C_OUT
    row_off = (base // C_OUT) * C_IN
    batch_tbl = table_hbm.at[pl.ds(row_off, C_IN)]
    pltpu.async_copy(idx_hbm.at[jstart // ROWS_PER_TILE], idx2, isem).wait()

    gathers = [None] * NBUF
    scatters = [None] * NBUF

    for p in range(3):
        gathers[p] = pltpu.async_copy(
            batch_tbl.at[idx2.at[p]], bufs[p], gsems[p])
    for c in range(NCHUNK):
        nxt = c + 3
        if nxt < NCHUNK:
            sn = nxt % NBUF
            if scatters[sn] is not None:
                scatters[sn].wait()
                scatters[sn] = None
            gathers[sn] = pltpu.async_copy(
                batch_tbl.at[idx2.at[nxt]], bufs[sn], gsems[sn])
        s = c % NBUF
        gathers[s].wait()
        scatters[s] = pltpu.async_copy(
            bufs[s], out_hbm.at[pl.ds(base + c * CHUNK, CHUNK)], ssems[s])
    for s in range(NBUF):
        if scatters[s] is not None:
            scatters[s].wait()


def kernel(x, indices):
    table = x.reshape(B * C_IN, D)
    out = _sc_gather(
        table, indices.reshape(C_OUT // ROWS_PER_TILE, NCHUNK, CHUNK))
    return out.reshape(B, C_OUT, D)
